# baseline (speedup 1.0000x reference)
import functools

import jax
import jax.numpy as jnp
from jax import lax
from jax.experimental import pallas as pl
from jax.experimental.pallas import tpu as pltpu
from jax.experimental.pallas import tpu_sc as plsc

F32 = jnp.float32
N = 10000
E = 320000
D = 128
DE = 16
HID = 64
L = 4
G = 64
NS_NODES = 10000
ES = 160000

K = 128
SB = 4
NSC = 2
NSUB = 16
NW = NSC * NSUB
CPT = 80
CPT_S = 40
NCH = NW * CPT
NCH_S = NW * CPT_S
E_PAD = NCH * K
ES_PAD = NCH_S * K
NROWS = 10112
RPT = NROWS // NSUB


def _sc_edge_pass(hw, ep, src2d, dst2d, cpt, with_ep):
    nsteps = cpt // SB
    mesh = plsc.VectorSubcoreMesh(core_axis_name="c", subcore_axis_name="s")

    scratch = [
        pltpu.VMEM((cpt, K), jnp.int32),
        pltpu.VMEM((cpt, K), jnp.int32),
        pltpu.VMEM((SB * K, HID), F32),
    ]
    if with_ep:
        scratch.append(pltpu.VMEM((SB * K // 2, 2 * HID), F32))
    scratch += [
        pltpu.VMEM_SHARED((NROWS, HID), F32),
        pltpu.SemaphoreType.DMA,
        pltpu.SemaphoreType.DMA,
    ]

    def body(*refs):
        if with_ep:
            (hw_hbm, ep_hbm, src_hbm, dst_hbm, out_hbm,
             src_all, dst_all, rows_v, ep_v, agg_sh, sem_g, sem_e) = refs
        else:
            (hw_hbm, src_hbm, dst_hbm, out_hbm,
             src_all, dst_all, rows_v, agg_sh, sem_g, sem_e) = refs
        m_v = rows_v
        c = lax.axis_index("c")
        s = lax.axis_index("s")
        w = c * NSUB + s

        def zero_row(i, _):
            for jj in range(0, HID, 16):
                m_v[i, pl.ds(jj, 16)] = jnp.zeros((16,), F32)
            return 0
        lax.fori_loop(0, SB * K, zero_row, 0)
        base = s * RPT
        pltpu.sync_copy(m_v, agg_sh.at[pl.ds(base, SB * K)])
        rem = RPT - SB * K
        pltpu.sync_copy(m_v.at[pl.ds(0, rem)], agg_sh.at[pl.ds(base + SB * K, rem)])
        plsc.subcore_barrier()

        pltpu.sync_copy(src_hbm.at[pl.ds(w * cpt, cpt)], src_all)
        pltpu.sync_copy(dst_hbm.at[pl.ds(w * cpt, cpt)], dst_all)

        def step(t, _):
            q0 = w * cpt + t * SB
            descs = []
            for b in range(SB):
                descs.append(pltpu.async_copy(
                    hw_hbm.at[src_all.at[t * SB + b]],
                    rows_v.at[pl.ds(b * K, K)], sem_g))
            if with_ep:
                ep_desc = pltpu.async_copy(
                    ep_hbm.at[pl.ds(q0 * (K // 2), SB * K // 2)], ep_v, sem_e)
            for d in descs:
                d.wait()
            if with_ep:
                ep_desc.wait()

                def mk_combine(h):
                    off = h * (SB * K // 2)

                    def combine(p, _):
                        for jj in range(0, HID, 16):
                            m_v[p + off, pl.ds(jj, 16)] = jnp.maximum(
                                rows_v[p + off, pl.ds(jj, 16)]
                                + ep_v[p, pl.ds(h * HID + jj, 16)],
                                jnp.zeros((16,), F32))
                        return 0
                    return combine
                lax.fori_loop(0, SB * K // 2, mk_combine(0), 0)
                lax.fori_loop(0, SB * K // 2, mk_combine(1), 0)
            for b in range(SB):
                pltpu.sync_copy(m_v.at[pl.ds(b * K, K)],
                                agg_sh.at[dst_all.at[t * SB + b]], add=True)
            return 0
        lax.fori_loop(0, nsteps, step, 0)

        plsc.subcore_barrier()
        pltpu.sync_copy(agg_sh.at[pl.ds(base, RPT)],
                        out_hbm.at[c, pl.ds(base, RPT)])

    run = functools.partial(
        pl.kernel,
        out_type=jax.ShapeDtypeStruct((NSC, NROWS, HID), F32),
        mesh=mesh,
        scratch_types=scratch,
        compiler_params=pltpu.CompilerParams(use_tc_tiling_on_sc=False),
    )(body)
    if with_ep:
        return run(hw, ep, src2d, dst2d)
    return run(hw, src2d, dst2d)


def _embed_body(nf_ref, snf_ref, wn_ref, bn_ref, a0_ref, ws_ref, bs_ref,
                h_ref, hw_ref, hs_ref):
    h0 = jnp.dot(nf_ref[...], wn_ref[...], preferred_element_type=F32) + bn_ref[...]
    h_ref[...] = h0
    hw_ref[...] = jnp.dot(h0, a0_ref[...], preferred_element_type=F32)
    hs_ref[...] = jnp.maximum(
        jnp.dot(snf_ref[...], ws_ref[...], preferred_element_type=F32) + bs_ref[...],
        0.0)


EP_BE = 2560
BAND = SB * K


def _ep_body(ef_ref, we_ref, wb_ref, be_ref, bm_ref, *out_refs):
    ef = ef_ref[...]
    half = BAND // 2
    for l in range(L):
        wb = wb_ref[l]
        m = jnp.dot(we_ref[...], wb, preferred_element_type=F32)
        cst = jnp.dot(be_ref[...], wb, preferred_element_type=F32) + bm_ref[l]
        tmp = jnp.dot(ef, m, preferred_element_type=F32) + cst
        for b in range(EP_BE // BAND):
            r0 = b * half
            out_refs[l][r0:r0 + half, :HID] = tmp[b * BAND:b * BAND + half, :]
            out_refs[l][r0:r0 + half, HID:] = tmp[b * BAND + half:(b + 1) * BAND, :]


def _make_update(need_hw):
    def body(*refs):
        if need_hw:
            (h_ref, p_ref, ut_ref, ub_ref, b_ref, a_ref, ho_ref, hwo_ref) = refs
        else:
            (h_ref, p_ref, ut_ref, ub_ref, b_ref, ho_ref) = refs
        h = h_ref[...]
        agg = p_ref[0, :N // 2, :] + p_ref[1, :N // 2, :]
        u = jnp.maximum(
            jnp.dot(h, ut_ref[...], preferred_element_type=F32)
            + jnp.dot(agg, ub_ref[...], preferred_element_type=F32)
            + b_ref[...], 0.0)
        hn = h + u
        ho_ref[...] = hn
        if need_hw:
            hwo_ref[...] = jnp.dot(hn, a_ref[...], preferred_element_type=F32)
    return body


def _final_body(h_ref, hs0_ref, ps_ref, ohp_ref, ohsp_ref, wl_ref, bl_ref,
                wot_ref, wos_ref, bo_ref, out_ref):
    h = h_ref[...]
    hs = hs0_ref[...] + ps_ref[0, :N // 2, :] + ps_ref[1, :N // 2, :]
    m1 = lax.dot_general(ohp_ref[...], h, (((0,), (0,)), ((), ())),
                         preferred_element_type=F32)
    g = m1[:G, :HID] + m1[G:, HID:]
    m2 = lax.dot_general(ohsp_ref[...], hs, (((0,), (0,)), ((), ())),
                         preferred_element_type=F32)
    gs = m2[:G, :HID] + m2[G:, HID:]
    for l in range(2):
        g = jnp.maximum(
            jnp.dot(g, wl_ref[l], preferred_element_type=F32) + bl_ref[l], 0.0)
    out_ref[...] = (jnp.dot(g, wot_ref[...], preferred_element_type=F32)
                    + jnp.dot(gs, wos_ref[...], preferred_element_type=F32)
                    + bo_ref[...])


def kernel(node_feats, edge_feats, edge_index, batch_ids, solv_node_feats,
           solv_edge_index, solv_batch_ids, W_node, b_node, W_edge, b_edge,
           W_msg, b_msg, W_upd, b_upd, W_lin, b_lin, W_solv, b_solv,
           W_out, b_out):
    src = edge_index[0].astype(jnp.int32)
    dst = edge_index[1].astype(jnp.int32)
    pad = E_PAD - E
    pad_src = (jnp.arange(pad, dtype=jnp.int32) * 37) % N
    pad_dst = N + (jnp.arange(pad, dtype=jnp.int32) % (NROWS - N))
    src2d = jnp.concatenate([src, pad_src]).reshape(NCH, K)
    dst2d = jnp.concatenate([dst, pad_dst]).reshape(NCH, K)

    s_src = solv_edge_index[0].astype(jnp.int32)
    s_dst = solv_edge_index[1].astype(jnp.int32)
    pad_s = ES_PAD - ES
    pad_ssrc = (jnp.arange(pad_s, dtype=jnp.int32) * 37) % NS_NODES
    pad_sdst = N + (jnp.arange(pad_s, dtype=jnp.int32) % (NROWS - N))
    ssrc2d = jnp.concatenate([s_src, pad_ssrc]).reshape(NCH_S, K)
    sdst2d = jnp.concatenate([s_dst, pad_sdst]).reshape(NCH_S, K)

    def bd(w):
        z = jnp.zeros_like(w)
        return jnp.concatenate(
            [jnp.concatenate([w, z], axis=1),
             jnp.concatenate([z, w], axis=1)], axis=0)

    a_all = W_msg[:, :HID, :]
    wb_all = W_msg[:, HID:, :]
    a2 = [bd(a_all[l]) for l in range(L)]
    ut2 = [bd(W_upd[l, :HID, :]) for l in range(L)]
    ub2 = [bd(W_upd[l, HID:, :]) for l in range(L)]
    bu2 = [jnp.concatenate([b_upd[l], b_upd[l]]).reshape(1, 2 * HID)
           for l in range(L)]
    NP = N // 2
    NR2 = NROWS // 2

    h, hw, hs0 = pl.pallas_call(
        _embed_body,
        out_shape=[jax.ShapeDtypeStruct((NP, 2 * HID), F32),
                   jax.ShapeDtypeStruct((NP, 2 * HID), F32),
                   jax.ShapeDtypeStruct((NP, 2 * HID), F32)],
    )(node_feats.reshape(NP, 2 * D), solv_node_feats.reshape(NP, 2 * D),
      bd(W_node), jnp.concatenate([b_node, b_node]).reshape(1, 2 * HID),
      a2[0], bd(W_solv), jnp.concatenate([b_solv, b_solv]).reshape(1, 2 * HID))

    EPH = E_PAD // 2
    nreal = E // EP_BE - 1
    ep_all = pl.pallas_call(
        _ep_body,
        grid=(EPH // (EP_BE // 2),),
        in_specs=[pl.BlockSpec((EP_BE, DE), lambda i: (jnp.minimum(i, nreal), 0)),
                  pl.BlockSpec((DE, HID), lambda i: (0, 0)),
                  pl.BlockSpec((L, HID, HID), lambda i: (0, 0, 0)),
                  pl.BlockSpec((1, HID), lambda i: (0, 0)),
                  pl.BlockSpec((L, 1, HID), lambda i: (0, 0, 0))],
        out_specs=[pl.BlockSpec((EP_BE // 2, 2 * HID), lambda i: (i, 0))] * L,
        out_shape=[jax.ShapeDtypeStruct((EPH, 2 * HID), F32)] * L,
    )(edge_feats, W_edge, wb_all, b_edge.reshape(1, HID),
      b_msg.reshape(L, 1, HID))

    for l in range(L):
        part = _sc_edge_pass(hw.reshape(N, HID), ep_all[l], src2d, dst2d,
                             CPT, True)
        need_hw = l < L - 1
        outs = [jax.ShapeDtypeStruct((NP, 2 * HID), F32)]
        args = [h, part.reshape(NSC, NR2, 2 * HID), ut2[l], ub2[l], bu2[l]]
        if need_hw:
            outs.append(jax.ShapeDtypeStruct((NP, 2 * HID), F32))
            args.append(a2[l + 1])
        res = pl.pallas_call(_make_update(need_hw), out_shape=outs)(*args)
        if need_hw:
            h, hw = res
        else:
            h = res[0]

    ps = _sc_edge_pass(hs0.reshape(N, HID), None, ssrc2d, sdst2d, CPT_S, False)

    garange = jnp.arange(G, dtype=jnp.int32)[None, :]
    ohp = (batch_ids.astype(jnp.int32)[:, None] == garange).astype(F32)
    ohsp = (solv_batch_ids.astype(jnp.int32)[:, None] == garange).astype(F32)
    out = pl.pallas_call(
        _final_body,
        out_shape=jax.ShapeDtypeStruct((G, 1), F32),
    )(h, hs0, ps.reshape(NSC, NR2, 2 * HID), ohp.reshape(NP, 2 * G),
      ohsp.reshape(NP, 2 * G),
      W_lin, b_lin.reshape(2, 1, HID), W_out[:HID], W_out[HID:],
      b_out.reshape(1, 1))
    return out


# speedup vs baseline: 1.5142x; 1.5142x over previous
import functools

import jax
import jax.numpy as jnp
from jax import lax
from jax.experimental import pallas as pl
from jax.experimental.pallas import tpu as pltpu
from jax.experimental.pallas import tpu_sc as plsc

F32 = jnp.float32
N = 10000
E = 320000
D = 128
DE = 16
HID = 64
L = 4
G = 64
NS_NODES = 10000
ES = 160000

K = 128
SB = 4
NSC = 2
NSUB = 16
NW = NSC * NSUB
CPT = 80
CPT_S = 40
NCH = NW * CPT
NCH_S = NW * CPT_S
E_PAD = NCH * K
ES_PAD = NCH_S * K
NROWS = 10112
RPT = NROWS // NSUB


def _sc_edge_pass(hw, ep, src2d, dst2d, cpt, with_ep):
    nsteps = cpt // SB
    mesh = plsc.VectorSubcoreMesh(core_axis_name="c", subcore_axis_name="s")

    scratch = [
        pltpu.VMEM((cpt, K), jnp.int32),
        pltpu.VMEM((cpt, K), jnp.int32),
        pltpu.VMEM((SB * K, HID), F32),
    ]
    if with_ep:
        scratch.append(pltpu.VMEM((SB * K // 2, 2 * HID), F32))
    scratch += [
        pltpu.VMEM_SHARED((NROWS, HID), F32),
        pltpu.SemaphoreType.DMA,
        pltpu.SemaphoreType.DMA,
    ]

    def body(*refs):
        if with_ep:
            (hw_hbm, ep_hbm, src_hbm, dst_hbm, out_hbm,
             src_all, dst_all, rows_v, ep_v, agg_sh, sem_g, sem_e) = refs
        else:
            (hw_hbm, src_hbm, dst_hbm, out_hbm,
             src_all, dst_all, rows_v, agg_sh, sem_g, sem_e) = refs
        m_v = rows_v
        c = lax.axis_index("c")
        s = lax.axis_index("s")
        w = c * NSUB + s

        def zero_row(i, _):
            for jj in range(0, HID, 16):
                m_v[i, pl.ds(jj, 16)] = jnp.zeros((16,), F32)
            return 0
        lax.fori_loop(0, SB * K, zero_row, 0)
        base = s * RPT
        pltpu.sync_copy(m_v, agg_sh.at[pl.ds(base, SB * K)])
        rem = RPT - SB * K
        pltpu.sync_copy(m_v.at[pl.ds(0, rem)], agg_sh.at[pl.ds(base + SB * K, rem)])
        plsc.subcore_barrier()

        pltpu.sync_copy(src_hbm.at[pl.ds(w * cpt, cpt)], src_all)
        pltpu.sync_copy(dst_hbm.at[pl.ds(w * cpt, cpt)], dst_all)

        def step(t, _):
            q0 = w * cpt + t * SB
            descs = []
            for b in range(SB):
                descs.append(pltpu.async_copy(
                    hw_hbm.at[src_all.at[t * SB + b]],
                    rows_v.at[pl.ds(b * K, K)], sem_g))
            if with_ep:
                ep_desc = pltpu.async_copy(
                    ep_hbm.at[pl.ds(q0 * (K // 2), SB * K // 2)], ep_v, sem_e)
            for d in descs:
                d.wait()
            if with_ep:
                ep_desc.wait()

                def combine(p, _):
                    for h in range(2):
                        for jj in range(0, HID, 16):
                            m_v[2 * p + h, pl.ds(jj, 16)] = jnp.maximum(
                                rows_v[2 * p + h, pl.ds(jj, 16)]
                                + ep_v[p, pl.ds(h * HID + jj, 16)],
                                jnp.zeros((16,), F32))
                    return 0
                lax.fori_loop(0, SB * K // 2, combine, 0)
            for b in range(SB):
                pltpu.sync_copy(m_v.at[pl.ds(b * K, K)],
                                agg_sh.at[dst_all.at[t * SB + b]], add=True)
            return 0
        lax.fori_loop(0, nsteps, step, 0)

        plsc.subcore_barrier()
        pltpu.sync_copy(agg_sh.at[pl.ds(base, RPT)],
                        out_hbm.at[c, pl.ds(base, RPT)])

    run = functools.partial(
        pl.kernel,
        out_type=jax.ShapeDtypeStruct((NSC, NROWS, HID), F32),
        mesh=mesh,
        scratch_types=scratch,
        compiler_params=pltpu.CompilerParams(use_tc_tiling_on_sc=False),
    )(body)
    if with_ep:
        return run(hw, ep, src2d, dst2d)
    return run(hw, src2d, dst2d)


def _embed_body(nf_ref, snf_ref, wn_ref, bn_ref, a0_ref, ws_ref, bs_ref,
                h_ref, hw_ref, hs_ref):
    h0 = jnp.dot(nf_ref[...], wn_ref[...], preferred_element_type=F32) + bn_ref[...]
    h_ref[...] = h0
    hw_ref[...] = jnp.dot(h0, a0_ref[...], preferred_element_type=F32)
    hs_ref[...] = jnp.maximum(
        jnp.dot(snf_ref[...], ws_ref[...], preferred_element_type=F32) + bs_ref[...],
        0.0)


def _ep_body(ef_ref, we_ref, wb_ref, be_ref, bm_ref, *out_refs):
    ef = ef_ref[...]
    z = jnp.zeros((DE, HID), F32)
    for l in range(L):
        wb = wb_ref[l]
        m = jnp.dot(we_ref[...], wb, preferred_element_type=F32)
        m2 = jnp.concatenate(
            [jnp.concatenate([m, z], axis=1),
             jnp.concatenate([z, m], axis=1)], axis=0)
        cst = jnp.dot(be_ref[...], wb, preferred_element_type=F32) + bm_ref[l]
        cst2 = jnp.concatenate([cst, cst], axis=1)
        out_refs[l][...] = jnp.dot(ef, m2, preferred_element_type=F32) + cst2


def _make_update(need_hw):
    def body(*refs):
        if need_hw:
            (h_ref, p_ref, ut_ref, ub_ref, b_ref, a_ref, ho_ref, hwo_ref) = refs
        else:
            (h_ref, p_ref, ut_ref, ub_ref, b_ref, ho_ref) = refs
        h = h_ref[...]
        agg = p_ref[0, :N // 2, :] + p_ref[1, :N // 2, :]
        u = jnp.maximum(
            jnp.dot(h, ut_ref[...], preferred_element_type=F32)
            + jnp.dot(agg, ub_ref[...], preferred_element_type=F32)
            + b_ref[...], 0.0)
        hn = h + u
        ho_ref[...] = hn
        if need_hw:
            hwo_ref[...] = jnp.dot(hn, a_ref[...], preferred_element_type=F32)
    return body


def _final_body(h_ref, hs0_ref, ps_ref, ohp_ref, ohsp_ref, wl_ref, bl_ref,
                wot_ref, wos_ref, bo_ref, out_ref):
    h = h_ref[...]
    hs = hs0_ref[...] + ps_ref[0, :N // 2, :] + ps_ref[1, :N // 2, :]
    m1 = lax.dot_general(ohp_ref[...], h, (((0,), (0,)), ((), ())),
                         preferred_element_type=F32)
    g = m1[:G, :HID] + m1[G:, HID:]
    m2 = lax.dot_general(ohsp_ref[...], hs, (((0,), (0,)), ((), ())),
                         preferred_element_type=F32)
    gs = m2[:G, :HID] + m2[G:, HID:]
    for l in range(2):
        g = jnp.maximum(
            jnp.dot(g, wl_ref[l], preferred_element_type=F32) + bl_ref[l], 0.0)
    out_ref[...] = (jnp.dot(g, wot_ref[...], preferred_element_type=F32)
                    + jnp.dot(gs, wos_ref[...], preferred_element_type=F32)
                    + bo_ref[...])


def kernel(node_feats, edge_feats, edge_index, batch_ids, solv_node_feats,
           solv_edge_index, solv_batch_ids, W_node, b_node, W_edge, b_edge,
           W_msg, b_msg, W_upd, b_upd, W_lin, b_lin, W_solv, b_solv,
           W_out, b_out):
    src = edge_index[0].astype(jnp.int32)
    dst = edge_index[1].astype(jnp.int32)
    pad = E_PAD - E
    pad_src = (jnp.arange(pad, dtype=jnp.int32) * 37) % N
    pad_dst = N + (jnp.arange(pad, dtype=jnp.int32) % (NROWS - N))
    src2d = jnp.concatenate([src, pad_src]).reshape(NCH, K)
    dst2d = jnp.concatenate([dst, pad_dst]).reshape(NCH, K)

    s_src = solv_edge_index[0].astype(jnp.int32)
    s_dst = solv_edge_index[1].astype(jnp.int32)
    pad_s = ES_PAD - ES
    pad_ssrc = (jnp.arange(pad_s, dtype=jnp.int32) * 37) % NS_NODES
    pad_sdst = N + (jnp.arange(pad_s, dtype=jnp.int32) % (NROWS - N))
    ssrc2d = jnp.concatenate([s_src, pad_ssrc]).reshape(NCH_S, K)
    sdst2d = jnp.concatenate([s_dst, pad_sdst]).reshape(NCH_S, K)

    def bd(w):
        z = jnp.zeros_like(w)
        return jnp.concatenate(
            [jnp.concatenate([w, z], axis=1),
             jnp.concatenate([z, w], axis=1)], axis=0)

    a_all = W_msg[:, :HID, :]
    wb_all = W_msg[:, HID:, :]
    a2 = [bd(a_all[l]) for l in range(L)]
    ut2 = [bd(W_upd[l, :HID, :]) for l in range(L)]
    ub2 = [bd(W_upd[l, HID:, :]) for l in range(L)]
    bu2 = [jnp.concatenate([b_upd[l], b_upd[l]]).reshape(1, 2 * HID)
           for l in range(L)]
    NP = N // 2
    NR2 = NROWS // 2

    h, hw, hs0 = pl.pallas_call(
        _embed_body,
        out_shape=[jax.ShapeDtypeStruct((NP, 2 * HID), F32),
                   jax.ShapeDtypeStruct((NP, 2 * HID), F32),
                   jax.ShapeDtypeStruct((NP, 2 * HID), F32)],
    )(node_feats.reshape(NP, 2 * D), solv_node_feats.reshape(NP, 2 * D),
      bd(W_node), jnp.concatenate([b_node, b_node]).reshape(1, 2 * HID),
      a2[0], bd(W_solv), jnp.concatenate([b_solv, b_solv]).reshape(1, 2 * HID))

    BE = 2048
    EPH = E_PAD // 2
    ef_pair = jnp.concatenate(
        [edge_feats, jnp.zeros((pad, DE), F32)], axis=0).reshape(EPH, 2 * DE)
    ep_all = pl.pallas_call(
        _ep_body,
        grid=(EPH // BE,),
        in_specs=[pl.BlockSpec((BE, 2 * DE), lambda i: (i, 0)),
                  pl.BlockSpec((DE, HID), lambda i: (0, 0)),
                  pl.BlockSpec((L, HID, HID), lambda i: (0, 0, 0)),
                  pl.BlockSpec((1, HID), lambda i: (0, 0)),
                  pl.BlockSpec((L, 1, HID), lambda i: (0, 0, 0))],
        out_specs=[pl.BlockSpec((BE, 2 * HID), lambda i: (i, 0))] * L,
        out_shape=[jax.ShapeDtypeStruct((EPH, 2 * HID), F32)] * L,
    )(ef_pair, W_edge, wb_all, b_edge.reshape(1, HID),
      b_msg.reshape(L, 1, HID))

    for l in range(L):
        part = _sc_edge_pass(hw.reshape(N, HID), ep_all[l], src2d, dst2d,
                             CPT, True)
        need_hw = l < L - 1
        outs = [jax.ShapeDtypeStruct((NP, 2 * HID), F32)]
        args = [h, part.reshape(NSC, NR2, 2 * HID), ut2[l], ub2[l], bu2[l]]
        if need_hw:
            outs.append(jax.ShapeDtypeStruct((NP, 2 * HID), F32))
            args.append(a2[l + 1])
        res = pl.pallas_call(_make_update(need_hw), out_shape=outs)(*args)
        if need_hw:
            h, hw = res
        else:
            h = res[0]

    ps = _sc_edge_pass(hs0.reshape(N, HID), None, ssrc2d, sdst2d, CPT_S, False)

    garange = jnp.arange(G, dtype=jnp.int32)[None, :]
    ohp = (batch_ids.astype(jnp.int32)[:, None] == garange).astype(F32)
    ohsp = (solv_batch_ids.astype(jnp.int32)[:, None] == garange).astype(F32)
    out = pl.pallas_call(
        _final_body,
        out_shape=jax.ShapeDtypeStruct((G, 1), F32),
    )(h, hs0, ps.reshape(NSC, NR2, 2 * HID), ohp.reshape(NP, 2 * G),
      ohsp.reshape(NP, 2 * G),
      W_lin, b_lin.reshape(2, 1, HID), W_out[:HID], W_out[HID:],
      b_out.reshape(1, 1))
    return out
